# Initial kernel scaffold; baseline (speedup 1.0000x reference)
import jax
import jax.numpy as jnp
from jax.experimental import pallas as pl


def kernel(x, Wg, bg, W1, b1, W2, b2):
    raise NotImplementedError("write your pallas kernel here")


# speedup vs baseline: 1.2589x; 1.2589x over previous
import functools

import jax
import jax.numpy as jnp
from jax.experimental import pallas as pl
from jax.experimental.pallas import tpu as pltpu

EP = 128


def _router_body(x_ref, wg_ref, bg_ref, gp_ref, wfull_ref):
    logits = jax.lax.dot_general(
        x_ref[...], wg_ref[...], (((1,), (0,)), ((), ())),
        preferred_element_type=jnp.float32,
    ) + bg_ref[...]
    m = jnp.max(logits, axis=-1, keepdims=True)
    ex = jnp.exp(logits - m)
    p = ex / jnp.sum(ex, axis=-1, keepdims=True)
    gp_ref[...] = p

    lane = jax.lax.broadcasted_iota(jnp.int32, p.shape, 1)
    m1 = jnp.max(p, axis=-1, keepdims=True)
    idx1 = jnp.min(jnp.where(p == m1, lane, EP), axis=-1, keepdims=True)
    sel1 = lane == idx1
    p_m = jnp.where(sel1, -1.0, p)
    m2 = jnp.max(p_m, axis=-1, keepdims=True)
    idx2 = jnp.min(jnp.where(p_m == m2, lane, EP), axis=-1, keepdims=True)
    sel2 = lane == idx2
    e2 = jnp.exp(m2 - m1)
    q1 = 1.0 / (1.0 + e2)
    q2 = e2 / (1.0 + e2)
    wfull_ref[...] = jnp.where(sel1, q1, 0.0) + jnp.where(sel2, q2, 0.0)


def _ffn_body(x_ref, w1_ref, b1_ref, w2_ref, b2_ref, wf_ref, y_ref, *, tm):
    e = pl.program_id(0)
    t = pl.program_id(1)
    rows = pl.ds(t * tm, tm)
    xt = x_ref[rows, :]
    h = jax.lax.dot_general(
        xt, w1_ref[0], (((1,), (0,)), ((), ())),
        preferred_element_type=jnp.float32,
    ) + b1_ref[0]
    h = jnp.maximum(h, 0.0).astype(jnp.bfloat16)
    o = jax.lax.dot_general(
        h, w2_ref[0], (((1,), (0,)), ((), ())),
        preferred_element_type=jnp.float32,
    ) + b2_ref[0]
    lane = jax.lax.broadcasted_iota(jnp.int32, wf_ref.shape, 1)
    w = jnp.sum(jnp.where(lane == e, wf_ref[...], 0.0), axis=-1, keepdims=True)
    contrib = w * o

    @pl.when(e == 0)
    def _():
        y_ref[rows, :] = contrib

    @pl.when(e != 0)
    def _():
        y_ref[rows, :] = y_ref[rows, :] + contrib


def kernel(x, Wg, bg, W1, b1, W2, b2):
    B, S, D = x.shape
    E = Wg.shape[1]
    T = B * S
    TM = 512 if T % 512 == 0 else T
    NT = T // TM

    xf = x.reshape(T, D)
    wgp = jnp.pad(Wg, ((0, 0), (0, EP - E)))
    bgp = jnp.pad(bg, (0, EP - E), constant_values=-1e30).reshape(1, EP)

    gate_prob_pad, wfull = pl.pallas_call(
        _router_body,
        grid=(NT,),
        in_specs=[
            pl.BlockSpec((TM, D), lambda t: (t, 0)),
            pl.BlockSpec((D, EP), lambda t: (0, 0)),
            pl.BlockSpec((1, EP), lambda t: (0, 0)),
        ],
        out_specs=[
            pl.BlockSpec((TM, EP), lambda t: (t, 0)),
            pl.BlockSpec((TM, EP), lambda t: (t, 0)),
        ],
        out_shape=[
            jax.ShapeDtypeStruct((T, EP), jnp.float32),
            jax.ShapeDtypeStruct((T, EP), jnp.float32),
        ],
    )(xf, wgp, bgp)

    xb = xf.astype(jnp.bfloat16)
    w1b = W1.astype(jnp.bfloat16)
    w2b = W2.astype(jnp.bfloat16)

    y = pl.pallas_call(
        functools.partial(_ffn_body, tm=TM),
        grid=(E, NT),
        in_specs=[
            pl.BlockSpec((T, D), lambda e, t: (0, 0)),
            pl.BlockSpec((1, D, D), lambda e, t: (e, 0, 0)),
            pl.BlockSpec((1, 1, D), lambda e, t: (e, 0, 0)),
            pl.BlockSpec((1, D, D), lambda e, t: (e, 0, 0)),
            pl.BlockSpec((1, 1, D), lambda e, t: (e, 0, 0)),
            pl.BlockSpec((TM, EP), lambda e, t: (t, 0)),
        ],
        out_specs=pl.BlockSpec((T, D), lambda e, t: (0, 0)),
        out_shape=jax.ShapeDtypeStruct((T, D), jnp.float32),
        compiler_params=pltpu.CompilerParams(
            dimension_semantics=("arbitrary", "arbitrary"),
        ),
    )(xb, w1b, b1.reshape(E, 1, D), w2b, b2.reshape(E, 1, D), wfull)

    return y.reshape(B, S, D), gate_prob_pad[:, :E]


# speedup vs baseline: 1.2633x; 1.0036x over previous
import functools

import jax
import jax.numpy as jnp
from jax import lax
from jax.experimental import pallas as pl
from jax.experimental.pallas import tpu as pltpu
from jax.experimental.pallas import tpu_sc as plsc

EP = 128
TM = 512


def _cumsum0(a, n):
    d = 1
    while d < n:
        a = a + jnp.pad(a[:-d], ((d, 0), (0, 0)))
        d *= 2
    return a


def _cumlane(a, n):
    d = 1
    while d < n:
        a = a + jnp.pad(a[:, :-d], ((0, 0), (d, 0)))
        d *= 2
    return a


def _router_body(x_ref, wg_ref, bg_ref,
                 gp_ref, pos0_ref, pos1_ref, q0_ref, q1_ref, te_ref, na_ref,
                 *, t, ntiles):
    logits = jax.lax.dot_general(
        x_ref[...], wg_ref[...], (((1,), (0,)), ((), ())),
        preferred_element_type=jnp.float32,
    ) + bg_ref[...]
    m = jnp.max(logits, axis=-1, keepdims=True)
    ex = jnp.exp(logits - m)
    p = ex / jnp.sum(ex, axis=-1, keepdims=True)
    gp_ref[...] = p

    lane = jax.lax.broadcasted_iota(jnp.int32, p.shape, 1)
    m1 = jnp.max(p, axis=-1, keepdims=True)
    idx1 = jnp.min(jnp.where(p == m1, lane, EP), axis=-1, keepdims=True)
    sel1 = lane == idx1
    p_m = jnp.where(sel1, -1.0, p)
    m2 = jnp.max(p_m, axis=-1, keepdims=True)
    idx2 = jnp.min(jnp.where(p_m == m2, lane, EP), axis=-1, keepdims=True)
    sel2 = lane == idx2
    e2 = jnp.exp(m2 - m1)
    qa = 1.0 / (1.0 + e2)
    qb = e2 / (1.0 + e2)
    q0_ref[...] = jnp.broadcast_to(qa, (t, EP))
    q1_ref[...] = jnp.broadcast_to(qb, (t, EP))

    s1 = sel1.astype(jnp.int32)
    s2 = sel2.astype(jnp.int32)
    c1 = _cumsum0(s1, t)
    c2 = _cumsum0(s2, t)
    tot1 = c1[t - 1:t, :]
    counts = tot1 + c2[t - 1:t, :]
    ntiles_e = (counts + (TM - 1)) // TM
    incl = _cumlane(ntiles_e, EP)
    excl = incl - ntiles_e
    offr = excl * TM
    pos0_ref[...] = jnp.sum(jnp.where(sel1, offr + c1 - 1, 0), axis=1,
                            keepdims=True).astype(jnp.int32)
    pos1_ref[...] = jnp.sum(jnp.where(sel2, offr + tot1 + c2 - 1, 0), axis=1,
                            keepdims=True).astype(jnp.int32)

    ntl = jnp.max(incl, axis=1, keepdims=True)
    na_ref[...] = ntl
    lane_t = jax.lax.broadcasted_iota(jnp.int32, (ntiles, EP), 1)
    icol = jax.lax.broadcasted_iota(jnp.int32, (ntiles, EP), 0)
    startb = jnp.broadcast_to(excl, (ntiles, EP))
    te_raw = jnp.max(jnp.where(startb <= icol, lane_t, 0), axis=1,
                     keepdims=True)
    lane_r = jax.lax.broadcasted_iota(jnp.int32, (1, EP), 1)
    mx = jnp.max(jnp.where(counts > 0, lane_r, 0), axis=1, keepdims=True)
    te_ref[...] = jnp.where(icol[:, :1] < ntl, te_raw, mx)


def _ffn_body(te_ref, na_ref, xs_ref, w1_ref, b1_ref, w2_ref, b2_ref, wv_ref,
              os_ref):
    i = pl.program_id(0)

    @pl.when(i < na_ref[0])
    def _():
        xt = xs_ref[...].astype(jnp.bfloat16)
        h = jax.lax.dot_general(
            xt, w1_ref[0], (((1,), (0,)), ((), ())),
            preferred_element_type=jnp.float32,
        ) + b1_ref[0]
        h = jnp.maximum(h, 0.0).astype(jnp.bfloat16)
        o = jax.lax.dot_general(
            h, w2_ref[0], (((1,), (0,)), ((), ())),
            preferred_element_type=jnp.float32,
        ) + b2_ref[0]
        os_ref[...] = o * wv_ref[:, 0:1]


def _scatter_body(x_hbm, pos0_hbm, pos1_hbm, q0_hbm, q1_hbm,
                  xs_hbm, wv_hbm,
                  idx0_v, idx1_v, xrows_v, q0_v, q1_v,
                  sem0, sem1, sem2, sem3, *, t, nw, nc):
    wid = lax.axis_index("s") * nc + lax.axis_index("c")
    tpw = t // nw
    ch = 32
    base = wid * tpw

    def chunk(c, carry):
        b = base + c * ch
        pltpu.sync_copy(pos0_hbm.at[pl.ds(b, ch)], idx0_v)
        pltpu.sync_copy(pos1_hbm.at[pl.ds(b, ch)], idx1_v)
        pltpu.sync_copy(x_hbm.at[pl.ds(b, ch), :], xrows_v)
        pltpu.sync_copy(q0_hbm.at[pl.ds(b, ch), :], q0_v)
        pltpu.sync_copy(q1_hbm.at[pl.ds(b, ch), :], q1_v)
        a0 = pltpu.async_copy(xrows_v, xs_hbm.at[idx0_v], sem0)
        a1 = pltpu.async_copy(xrows_v, xs_hbm.at[idx1_v], sem1)
        a2 = pltpu.async_copy(q0_v, wv_hbm.at[idx0_v], sem2)
        a3 = pltpu.async_copy(q1_v, wv_hbm.at[idx1_v], sem3)
        a0.wait()
        a1.wait()
        a2.wait()
        a3.wait()
        return carry

    lax.fori_loop(0, tpw // ch, chunk, 0)


def _combine_body(os_hbm, pos0_hbm, pos1_hbm, y_hbm,
                  idx0_v, idx1_v, r0_v, r1_v, yv_v, sem0, sem1,
                  *, t, d, nw, nc):
    wid = lax.axis_index("s") * nc + lax.axis_index("c")
    tpw = t // nw
    ch = 8
    base = wid * tpw

    def chunk(c, carry):
        tb = base + c * ch
        pltpu.sync_copy(pos0_hbm.at[pl.ds(tb, ch)], idx0_v)
        pltpu.sync_copy(pos1_hbm.at[pl.ds(tb, ch)], idx1_v)
        a0 = pltpu.async_copy(os_hbm.at[idx0_v], r0_v, sem0)
        a1 = pltpu.async_copy(os_hbm.at[idx1_v], r1_v, sem1)
        a0.wait()
        a1.wait()
        for i in range(ch):
            for j in range(d // 16):
                sl = pl.ds(j * 16, 16)
                yv_v[i, sl] = r0_v[i, sl] + r1_v[i, sl]
        pltpu.sync_copy(yv_v, y_hbm.at[pl.ds(tb, ch), :])
        return carry

    lax.fori_loop(0, tpw // ch, chunk, 0)


def kernel(x, Wg, bg, W1, b1, W2, b2):
    B, S, D = x.shape
    E = Wg.shape[1]
    T = B * S
    TOP = 2
    NTILES = (TOP * T) // TM + (E - 1)
    CAP = NTILES * TM

    xf = x.reshape(T, D)
    wgp = jnp.pad(Wg, ((0, 0), (0, EP - E)))
    bgp = jnp.pad(bg, (0, EP - E), constant_values=-1e30).reshape(1, EP)

    gp, pos0, pos1, q0r, q1r, te, na = pl.pallas_call(
        functools.partial(_router_body, t=T, ntiles=NTILES),
        grid=(1,),
        in_specs=[
            pl.BlockSpec((T, D), lambda i: (0, 0)),
            pl.BlockSpec((D, EP), lambda i: (0, 0)),
            pl.BlockSpec((1, EP), lambda i: (0, 0)),
        ],
        out_specs=[
            pl.BlockSpec((T, EP), lambda i: (0, 0)),
            pl.BlockSpec((T, 1), lambda i: (0, 0)),
            pl.BlockSpec((T, 1), lambda i: (0, 0)),
            pl.BlockSpec((T, EP), lambda i: (0, 0)),
            pl.BlockSpec((T, EP), lambda i: (0, 0)),
            pl.BlockSpec((NTILES, 1), lambda i: (0, 0)),
            pl.BlockSpec((1, 1), lambda i: (0, 0)),
        ],
        out_shape=[
            jax.ShapeDtypeStruct((T, EP), jnp.float32),
            jax.ShapeDtypeStruct((T, 1), jnp.int32),
            jax.ShapeDtypeStruct((T, 1), jnp.int32),
            jax.ShapeDtypeStruct((T, EP), jnp.float32),
            jax.ShapeDtypeStruct((T, EP), jnp.float32),
            jax.ShapeDtypeStruct((NTILES, 1), jnp.int32),
            jax.ShapeDtypeStruct((1, 1), jnp.int32),
        ],
    )(xf, wgp, bgp)

    pos0 = pos0.reshape(T)
    pos1 = pos1.reshape(T)

    xs, wv = _sc_scatter(xf, pos0, pos1, q0r, q1r, CAP)

    w1b = W1.astype(jnp.bfloat16)
    w2b = W2.astype(jnp.bfloat16)

    os_full = pl.pallas_call(
        _ffn_body,
        grid_spec=pltpu.PrefetchScalarGridSpec(
            num_scalar_prefetch=2,
            grid=(NTILES,),
            in_specs=[
                pl.BlockSpec((TM, D),
                             lambda i, te, na: (jnp.minimum(i, na[0] - 1), 0)),
                pl.BlockSpec((1, D, D), lambda i, te, na: (te[i], 0, 0)),
                pl.BlockSpec((1, 1, D), lambda i, te, na: (te[i], 0, 0)),
                pl.BlockSpec((1, D, D), lambda i, te, na: (te[i], 0, 0)),
                pl.BlockSpec((1, 1, D), lambda i, te, na: (te[i], 0, 0)),
                pl.BlockSpec((TM, EP),
                             lambda i, te, na: (jnp.minimum(i, na[0] - 1), 0)),
            ],
            out_specs=pl.BlockSpec(
                (TM, D),
                lambda i, te, na: (jnp.where(i < na[0], i, na[0]), 0)),
        ),
        out_shape=jax.ShapeDtypeStruct((CAP + TM, D), jnp.float32),
        compiler_params=pltpu.CompilerParams(
            dimension_semantics=("arbitrary",),
        ),
    )(te.reshape(NTILES), na.reshape(1), xs, w1b, b1.reshape(E, 1, D),
      w2b, b2.reshape(E, 1, D), wv)

    y = _sc_combine(os_full, pos0, pos1)

    return y.reshape(B, S, D), gp[:, :E]


def _sc_scatter(xf, pos0, pos1, q0r, q1r, cap):
    T, D = xf.shape
    info = plsc.get_sparse_core_info()
    NC, NS = info.num_cores, info.num_subcores
    mesh = plsc.VectorSubcoreMesh(core_axis_name="c", subcore_axis_name="s")
    scatter = pl.kernel(
        functools.partial(_scatter_body, t=T, nw=NC * NS, nc=NC),
        out_type=[
            jax.ShapeDtypeStruct((cap, D), jnp.float32),
            jax.ShapeDtypeStruct((cap, EP), jnp.float32),
        ],
        mesh=mesh,
        scratch_types=[
            pltpu.VMEM((32,), jnp.int32),
            pltpu.VMEM((32,), jnp.int32),
            pltpu.VMEM((32, D), jnp.float32),
            pltpu.VMEM((32, EP), jnp.float32),
            pltpu.VMEM((32, EP), jnp.float32),
            pltpu.SemaphoreType.DMA,
            pltpu.SemaphoreType.DMA,
            pltpu.SemaphoreType.DMA,
            pltpu.SemaphoreType.DMA,
        ],
    )
    return scatter(xf, pos0, pos1, q0r, q1r)


def _sc_combine(os_full, pos0, pos1):
    T = pos0.shape[0]
    D = os_full.shape[1]
    info = plsc.get_sparse_core_info()
    NC, NS = info.num_cores, info.num_subcores
    mesh = plsc.VectorSubcoreMesh(core_axis_name="c", subcore_axis_name="s")
    combine = pl.kernel(
        functools.partial(_combine_body, t=T, d=D, nw=NC * NS, nc=NC),
        out_type=jax.ShapeDtypeStruct((T, D), jnp.float32),
        mesh=mesh,
        scratch_types=[
            pltpu.VMEM((8,), jnp.int32),
            pltpu.VMEM((8,), jnp.int32),
            pltpu.VMEM((8, D), jnp.float32),
            pltpu.VMEM((8, D), jnp.float32),
            pltpu.VMEM((8, D), jnp.float32),
            pltpu.SemaphoreType.DMA,
            pltpu.SemaphoreType.DMA,
        ],
    )
    return combine(os_full, pos0, pos1)
